# initial kernel scaffold (unmeasured)
import jax
import jax.numpy as jnp
from jax import lax
from jax.experimental import pallas as pl
from jax.experimental.pallas import tpu as pltpu

N_DEV = 4


def kernel(x, w_mat, scale_x, scale_w):
    m_per, k = x.shape
    _, n_total = w_mat.shape
    n_per = n_total // N_DEV
    m_total = m_per * N_DEV

    me = lax.axis_index("i")
    w_my = lax.dynamic_slice(w_mat, (0, me * n_per), (k, n_per))
    x8 = x.astype(jnp.float8_e4m3fn)
    w8 = w_my.astype(jnp.float8_e4m3fn)

    def body(x_ref, w_ref, sx_ref, sw_ref, out_ref,
             xg, stage, send_sems, recv_sems, copy_sem):
        my = lax.axis_index("i")
        left = lax.rem(my + (N_DEV - 1), N_DEV)
        right = lax.rem(my + 1, N_DEV)

        barrier = pltpu.get_barrier_semaphore()
        for nbr in (left, right):
            pl.semaphore_signal(barrier, inc=1, device_id=(nbr,),
                                device_id_type=pl.DeviceIdType.MESH)
        pl.semaphore_wait(barrier, 2)

        scale = sx_ref[0] * sw_ref[0]

        def compute_block(origin):
            a = xg[pl.ds(origin * m_per, m_per), :]
            acc = lax.dot_general(
                a, w_ref[...],
                dimension_numbers=(((1,), (0,)), ((), ())),
                preferred_element_type=jnp.float32,
            )
            stage[...] = jnp.maximum(acc * scale, 0.0)
            cp = pltpu.make_async_copy(
                stage, out_ref.at[pl.ds(origin * m_per, m_per), :], copy_sem)
            cp.start()
            cp.wait()

        xg[pl.ds(my * m_per, m_per), :] = x_ref[...]
        compute_block(my)

        for h in range(N_DEV - 1):
            src_o = lax.rem(my + (2 * N_DEV - h), N_DEV)
            rdma = pltpu.make_async_remote_copy(
                src_ref=xg.at[pl.ds(src_o * m_per, m_per), :],
                dst_ref=xg.at[pl.ds(src_o * m_per, m_per), :],
                send_sem=send_sems.at[h],
                recv_sem=recv_sems.at[h],
                device_id=(right,),
                device_id_type=pl.DeviceIdType.MESH,
            )
            rdma.start()
            rdma.wait()
            arrived = lax.rem(my + (2 * N_DEV - h - 1), N_DEV)
            compute_block(arrived)

    return pl.pallas_call(
        body,
        out_shape=jax.ShapeDtypeStruct((m_total, n_per), jnp.float32),
        in_specs=[
            pl.BlockSpec(memory_space=pltpu.VMEM),
            pl.BlockSpec(memory_space=pltpu.VMEM),
            pl.BlockSpec(memory_space=pltpu.SMEM),
            pl.BlockSpec(memory_space=pltpu.SMEM),
        ],
        out_specs=pl.BlockSpec(memory_space=pltpu.ANY),
        scratch_shapes=[
            pltpu.VMEM((m_total, k), jnp.float8_e4m3fn),
            pltpu.VMEM((m_per, n_per), jnp.float32),
            pltpu.SemaphoreType.DMA((N_DEV - 1,)),
            pltpu.SemaphoreType.DMA((N_DEV - 1,)),
            pltpu.SemaphoreType.DMA,
        ],
        compiler_params=pltpu.CompilerParams(collective_id=0),
    )(x8, w8, scale_x, scale_w)


# baseline (device time: 238828 ns/iter reference)
import jax
import jax.numpy as jnp
from jax import lax
from jax.experimental import pallas as pl
from jax.experimental.pallas import tpu as pltpu

N_DEV = 4


def kernel(x, w_mat, scale_x, scale_w):
    m_per, k = x.shape
    _, n_total = w_mat.shape
    n_per = n_total // N_DEV
    m_total = m_per * N_DEV

    me = lax.axis_index("i")
    w_my = lax.dynamic_slice(w_mat, (0, me * n_per), (k, n_per))
    x8 = x.astype(jnp.float8_e4m3fn)
    w8 = w_my.astype(jnp.float8_e4m3fn)

    def body(x_ref, w_ref, sx_ref, sw_ref, out_ref,
             xg, stage, send_sems, recv_sems, copy_sem):
        my = lax.axis_index("i")
        left = lax.rem(my + (N_DEV - 1), N_DEV)
        right = lax.rem(my + 1, N_DEV)

        barrier = pltpu.get_barrier_semaphore()
        for nbr in (left, right):
            pl.semaphore_signal(barrier, inc=1, device_id=(nbr,),
                                device_id_type=pl.DeviceIdType.MESH)
        pl.semaphore_wait(barrier, 2)

        scale = sx_ref[0] * sw_ref[0]

        def compute_block(origin):
            a = xg[pl.ds(origin * m_per, m_per), :]
            acc = lax.dot_general(
                a, w_ref[...],
                dimension_numbers=(((1,), (0,)), ((), ())),
                preferred_element_type=jnp.float32,
            )
            stage[...] = jnp.maximum(acc * scale, 0.0)
            cp = pltpu.make_async_copy(
                stage, out_ref.at[pl.ds(origin * m_per, m_per), :], copy_sem)
            cp.start()
            cp.wait()

        xg[pl.ds(my * m_per, m_per), :] = x_ref[...]
        compute_block(my)

        for h in range(N_DEV - 1):
            src_o = lax.rem(my + (2 * N_DEV - h), N_DEV)
            rdma = pltpu.make_async_remote_copy(
                src_ref=xg.at[pl.ds(src_o * m_per, m_per), :],
                dst_ref=xg.at[pl.ds(src_o * m_per, m_per), :],
                send_sem=send_sems.at[h],
                recv_sem=recv_sems.at[h],
                device_id=(right,),
                device_id_type=pl.DeviceIdType.MESH,
            )
            rdma.start()
            rdma.wait()
            arrived = lax.rem(my + (2 * N_DEV - h - 1), N_DEV)
            compute_block(arrived)

    return pl.pallas_call(
        body,
        out_shape=jax.ShapeDtypeStruct((m_total, n_per), jnp.float32),
        in_specs=[
            pl.BlockSpec(memory_space=pltpu.VMEM),
            pl.BlockSpec(memory_space=pltpu.VMEM),
            pl.BlockSpec(memory_space=pltpu.SMEM),
            pl.BlockSpec(memory_space=pltpu.SMEM),
        ],
        out_specs=pl.BlockSpec(memory_space=pl.ANY),
        scratch_shapes=[
            pltpu.VMEM((m_total, k), jnp.float8_e4m3fn),
            pltpu.VMEM((m_per, n_per), jnp.float32),
            pltpu.SemaphoreType.DMA((N_DEV - 1,)),
            pltpu.SemaphoreType.DMA((N_DEV - 1,)),
            pltpu.SemaphoreType.DMA,
        ],
        compiler_params=pltpu.CompilerParams(collective_id=0),
    )(x8, w8, scale_x, scale_w)


# device time: 132285 ns/iter; 1.8054x vs baseline; 1.8054x over previous
import jax
import jax.numpy as jnp
from jax import lax
from jax.experimental import pallas as pl
from jax.experimental.pallas import tpu as pltpu

N_DEV = 4


def kernel(x, w_mat, scale_x, scale_w):
    m_per, k = x.shape
    _, n_total = w_mat.shape
    n_per = n_total // N_DEV
    m_total = m_per * N_DEV
    half = m_per // 2

    me = lax.axis_index("i")
    w_my = lax.dynamic_slice(w_mat, (0, me * n_per), (k, n_per))
    x8 = x.astype(jnp.float8_e4m3fn)
    w8 = w_my.astype(jnp.float8_e4m3fn)

    def body(x_ref, w_ref, sx_ref, sw_ref, out_ref,
             xg, stage, rs_sems, rr_sems, ls_sems, lr_sems, copy_sems):
        my = lax.axis_index("i")
        left = lax.rem(my + (N_DEV - 1), N_DEV)
        right = lax.rem(my + 1, N_DEV)

        barrier = pltpu.get_barrier_semaphore()
        for nbr in (left, right):
            pl.semaphore_signal(barrier, inc=1, device_id=(nbr,),
                                device_id_type=pl.DeviceIdType.MESH)
        pl.semaphore_wait(barrier, 2)

        scale = sx_ref[0] * sw_ref[0]

        pending = [None, None]
        slot = [0]

        def compute_half(a_slice, out_row):
            s = slot[0] & 1
            slot[0] += 1
            if pending[s] is not None:
                pending[s].wait()
            acc = lax.dot_general(
                a_slice[...], w_ref[...],
                dimension_numbers=(((1,), (0,)), ((), ())),
                preferred_element_type=jnp.float32,
            )
            stage[s] = jnp.maximum(acc * scale, 0.0)
            cp = pltpu.make_async_copy(
                stage.at[s], out_ref.at[pl.ds(out_row, half), :],
                copy_sems.at[s])
            cp.start()
            pending[s] = cp

        def a_rows(o):
            return pl.ds(o * m_per, half)

        def b_rows(o):
            return pl.ds(o * m_per + half, half)

        for h in range(N_DEV - 1):
            o_r = lax.rem(my + (N_DEV - h), N_DEV)
            o_l = lax.rem(my + h, N_DEV)
            src_r = x_ref.at[pl.ds(0, half), :] if h == 0 else xg.at[a_rows(o_r), :]
            src_l = x_ref.at[pl.ds(half, half), :] if h == 0 else xg.at[b_rows(o_l), :]
            rdma_r = pltpu.make_async_remote_copy(
                src_ref=src_r, dst_ref=xg.at[a_rows(o_r), :],
                send_sem=rs_sems.at[h], recv_sem=rr_sems.at[h],
                device_id=(right,), device_id_type=pl.DeviceIdType.MESH)
            rdma_l = pltpu.make_async_remote_copy(
                src_ref=src_l, dst_ref=xg.at[b_rows(o_l), :],
                send_sem=ls_sems.at[h], recv_sem=lr_sems.at[h],
                device_id=(left,), device_id_type=pl.DeviceIdType.MESH)
            rdma_r.start()
            rdma_l.start()

            if h == 0:
                compute_half(x_ref.at[pl.ds(0, half), :], my * m_per)
                compute_half(x_ref.at[pl.ds(half, half), :], my * m_per + half)
            elif h == 1:
                o_a = lax.rem(my + (N_DEV - 1), N_DEV)
                o_b = lax.rem(my + 1, N_DEV)
                compute_half(xg.at[a_rows(o_a), :], o_a * m_per)
                compute_half(xg.at[b_rows(o_b), :], o_b * m_per + half)
            else:
                o2 = lax.rem(my + 2, N_DEV)
                compute_half(xg.at[a_rows(o2), :], o2 * m_per)
                compute_half(xg.at[b_rows(o2), :], o2 * m_per + half)

            rdma_r.wait_recv()
            rdma_l.wait_recv()
            rdma_r.wait_send()
            rdma_l.wait_send()

        o_a = lax.rem(my + 1, N_DEV)
        o_b = lax.rem(my + (N_DEV - 1), N_DEV)
        compute_half(xg.at[a_rows(o_a), :], o_a * m_per)
        compute_half(xg.at[b_rows(o_b), :], o_b * m_per + half)

        for p in pending:
            if p is not None:
                p.wait()

    return pl.pallas_call(
        body,
        out_shape=jax.ShapeDtypeStruct((m_total, n_per), jnp.float32),
        in_specs=[
            pl.BlockSpec(memory_space=pltpu.VMEM),
            pl.BlockSpec(memory_space=pltpu.VMEM),
            pl.BlockSpec(memory_space=pltpu.SMEM),
            pl.BlockSpec(memory_space=pltpu.SMEM),
        ],
        out_specs=pl.BlockSpec(memory_space=pl.ANY),
        scratch_shapes=[
            pltpu.VMEM((m_total, k), jnp.float8_e4m3fn),
            pltpu.VMEM((2, half, n_per), jnp.float32),
            pltpu.SemaphoreType.DMA((N_DEV - 1,)),
            pltpu.SemaphoreType.DMA((N_DEV - 1,)),
            pltpu.SemaphoreType.DMA((N_DEV - 1,)),
            pltpu.SemaphoreType.DMA((N_DEV - 1,)),
            pltpu.SemaphoreType.DMA((2,)),
        ],
        compiler_params=pltpu.CompilerParams(collective_id=0),
    )(x8, w8, scale_x, scale_w)


# device time: 115133 ns/iter; 2.0744x vs baseline; 1.1490x over previous
import jax
import jax.numpy as jnp
from jax import lax
from jax.experimental import pallas as pl
from jax.experimental.pallas import tpu as pltpu

N_DEV = 4
W_CHUNKS = 4


def kernel(x, w_mat, scale_x, scale_w):
    m_per, k = x.shape
    _, n_total = w_mat.shape
    n_per = n_total // N_DEV
    m_total = m_per * N_DEV
    half = m_per // 2
    kc = k // W_CHUNKS

    def body(x_ref, w_ref, sx_ref, sw_ref, out_ref,
             xg, w8, fstage, stage,
             fstage_sems, rs_sems, rr_sems, ls_sems, lr_sems, copy_sems):
        my = lax.axis_index("i")
        left = lax.rem(my + (N_DEV - 1), N_DEV)
        right = lax.rem(my + 1, N_DEV)

        barrier = pltpu.get_barrier_semaphore()
        for nbr in (left, right):
            pl.semaphore_signal(barrier, inc=1, device_id=(nbr,),
                                device_id_type=pl.DeviceIdType.MESH)
        pl.semaphore_wait(barrier, 2)

        scale = sx_ref[0] * sw_ref[0]

        def a_rows(o):
            return pl.ds(o * m_per, half)

        def b_rows(o):
            return pl.ds(o * m_per + half, half)

        xcps = []
        for c in range(2):
            cp = pltpu.make_async_copy(
                x_ref.at[:, pl.ds(c * n_per, n_per)],
                fstage.at[c], fstage_sems.at[c])
            cp.start()
            xcps.append(cp)
        for c in range(2):
            xcps[c].wait()
            xg[pl.ds(my * m_per, m_per), pl.ds(c * n_per, n_per)] = (
                fstage[c].astype(jnp.float8_e4m3fn))

        def start_hop(h):
            o_r = lax.rem(my + (N_DEV - h), N_DEV)
            o_l = lax.rem(my + h, N_DEV)
            rdma_r = pltpu.make_async_remote_copy(
                src_ref=xg.at[a_rows(o_r), :], dst_ref=xg.at[a_rows(o_r), :],
                send_sem=rs_sems.at[h], recv_sem=rr_sems.at[h],
                device_id=(right,), device_id_type=pl.DeviceIdType.MESH)
            rdma_l = pltpu.make_async_remote_copy(
                src_ref=xg.at[b_rows(o_l), :], dst_ref=xg.at[b_rows(o_l), :],
                send_sem=ls_sems.at[h], recv_sem=lr_sems.at[h],
                device_id=(left,), device_id_type=pl.DeviceIdType.MESH)
            rdma_r.start()
            rdma_l.start()
            return rdma_r, rdma_l

        hop = start_hop(0)

        wcps = [None] * W_CHUNKS

        def start_wchunk(c):
            cp = pltpu.make_async_copy(
                w_ref.at[pl.ds(c * kc, kc), pl.ds(my * n_per, n_per)],
                fstage.at[c % 2], fstage_sems.at[c % 2])
            cp.start()
            wcps[c] = cp

        start_wchunk(0)
        start_wchunk(1)
        for c in range(W_CHUNKS):
            wcps[c].wait()
            w8[pl.ds(c * kc, kc), :] = fstage[c % 2].astype(jnp.float8_e4m3fn)
            if c + 2 < W_CHUNKS:
                start_wchunk(c + 2)

        pending = [None, None]
        slot = [0]

        def compute_half(rows, out_row):
            s = slot[0] & 1
            slot[0] += 1
            if pending[s] is not None:
                pending[s].wait()
            acc = lax.dot_general(
                xg[rows, :], w8[...],
                dimension_numbers=(((1,), (0,)), ((), ())),
                preferred_element_type=jnp.float32,
            )
            stage[s] = jnp.maximum(acc * scale, 0.0)
            cp = pltpu.make_async_copy(
                stage.at[s], out_ref.at[pl.ds(out_row, half), :],
                copy_sems.at[s])
            cp.start()
            pending[s] = cp

        for h in range(N_DEV - 1):
            if h == 0:
                compute_half(a_rows(my), my * m_per)
                compute_half(b_rows(my), my * m_per + half)
            elif h == 1:
                o_a = lax.rem(my + (N_DEV - 1), N_DEV)
                o_b = lax.rem(my + 1, N_DEV)
                compute_half(a_rows(o_a), o_a * m_per)
                compute_half(b_rows(o_b), o_b * m_per + half)
            else:
                o2 = lax.rem(my + 2, N_DEV)
                compute_half(a_rows(o2), o2 * m_per)
                compute_half(b_rows(o2), o2 * m_per + half)

            rdma_r, rdma_l = hop
            rdma_r.wait_recv()
            rdma_l.wait_recv()
            rdma_r.wait_send()
            rdma_l.wait_send()
            if h + 1 < N_DEV - 1:
                hop = start_hop(h + 1)

        o_a = lax.rem(my + 1, N_DEV)
        o_b = lax.rem(my + (N_DEV - 1), N_DEV)
        compute_half(a_rows(o_a), o_a * m_per)
        compute_half(b_rows(o_b), o_b * m_per + half)

        for p in pending:
            if p is not None:
                p.wait()

    return pl.pallas_call(
        body,
        out_shape=jax.ShapeDtypeStruct((m_total, n_per), jnp.float32),
        in_specs=[
            pl.BlockSpec(memory_space=pl.ANY),
            pl.BlockSpec(memory_space=pl.ANY),
            pl.BlockSpec(memory_space=pltpu.SMEM),
            pl.BlockSpec(memory_space=pltpu.SMEM),
        ],
        out_specs=pl.BlockSpec(memory_space=pl.ANY),
        scratch_shapes=[
            pltpu.VMEM((m_total, k), jnp.float8_e4m3fn),
            pltpu.VMEM((k, n_per), jnp.float8_e4m3fn),
            pltpu.VMEM((2, m_per, n_per), jnp.float32),
            pltpu.VMEM((2, half, n_per), jnp.float32),
            pltpu.SemaphoreType.DMA((2,)),
            pltpu.SemaphoreType.DMA((N_DEV - 1,)),
            pltpu.SemaphoreType.DMA((N_DEV - 1,)),
            pltpu.SemaphoreType.DMA((N_DEV - 1,)),
            pltpu.SemaphoreType.DMA((N_DEV - 1,)),
            pltpu.SemaphoreType.DMA((2,)),
        ],
        compiler_params=pltpu.CompilerParams(
            collective_id=0, vmem_limit_bytes=100 * 1024 * 1024),
    )(x, w_mat, scale_x, scale_w)
